# Initial kernel scaffold; baseline (speedup 1.0000x reference)
import jax
import jax.numpy as jnp
from jax.experimental import pallas as pl


def kernel(x, edge_index1, edge_index2, W1, b1, W2, b2):
    raise NotImplementedError("write your pallas kernel here")


# speedup vs baseline: 61.2881x; 61.2881x over previous
import functools

import jax
import jax.numpy as jnp
from jax import lax
from jax.experimental import pallas as pl
from jax.experimental.pallas import tpu as pltpu
from jax.experimental.pallas import tpu_sc as plsc

N1, N2, D = 8192, 1024, 128
E1, E2 = 262144, 32768
NC, NS = 2, 16
NW = NC * NS
TRASH = N2
ACC_ROWS = N2 + NS


def _mesh():
    return plsc.VectorSubcoreMesh(
        core_axis_name="c", subcore_axis_name="s",
        num_cores=NC, num_subcores=NS)


def _deg_call(dst1_t, dst2_t, ones_t):
    c1, c2 = E1 // NW // 128, E2 // NW // 128
    r1, r2 = N1 // NS, N2 // NS

    @functools.partial(
        pl.kernel,
        out_type=[jax.ShapeDtypeStruct((NC, N1), jnp.float32),
                  jax.ShapeDtypeStruct((NC, N2), jnp.float32)],
        mesh=_mesh(),
        scratch_types=[pltpu.VMEM((c1, 128), jnp.int32),
                       pltpu.VMEM((c2, 128), jnp.int32),
                       pltpu.VMEM((c1, 128), jnp.float32),
                       pltpu.VMEM((r1,), jnp.float32),
                       pltpu.VMEM_SHARED((N1,), jnp.float32),
                       pltpu.VMEM_SHARED((N2,), jnp.float32)])
    def k(dst1_h, dst2_h, ones_h, deg1p_h, deg2p_h,
          d1v, d2v, onesv, stg, deg1_s, deg2_s):
        c = lax.axis_index("c")
        s = lax.axis_index("s")
        w = s * NC + c

        def zb(i, carry):
            stg[pl.ds(i * 16, 16)] = jnp.zeros((16,), jnp.float32)
            return carry
        lax.fori_loop(0, r1 // 16, zb, 0)
        pltpu.sync_copy(stg, deg1_s.at[pl.ds(s * r1, r1)])
        pltpu.sync_copy(stg.at[pl.ds(0, r2)], deg2_s.at[pl.ds(s * r2, r2)])
        pltpu.sync_copy(ones_h, onesv)
        pltpu.sync_copy(dst1_h.at[w], d1v)
        pltpu.sync_copy(dst2_h.at[w], d2v)
        plsc.subcore_barrier()

        def b1(j, carry):
            pltpu.sync_copy(onesv.at[j], deg1_s.at[d1v.at[j]], add=True)
            return carry
        lax.fori_loop(0, c1, b1, 0)

        def b2(j, carry):
            pltpu.sync_copy(onesv.at[j], deg2_s.at[d2v.at[j]], add=True)
            return carry
        lax.fori_loop(0, c2, b2, 0)
        plsc.subcore_barrier()
        pltpu.sync_copy(deg1_s.at[pl.ds(s * r1, r1)], stg)
        pltpu.sync_copy(stg, deg1p_h.at[c, pl.ds(s * r1, r1)])
        pltpu.sync_copy(deg2_s.at[pl.ds(s * r2, r2)], stg.at[pl.ds(0, r2)])
        pltpu.sync_copy(stg.at[pl.ds(0, r2)], deg2p_h.at[c, pl.ds(s * r2, r2)])

    return k(dst1_t, dst2_t, ones_t)


def _psum16(v):
    iot = lax.iota(jnp.int32, 16)
    out = v
    for k in (1, 2, 4, 8):
        idxk = jnp.maximum(iot - k, 0)
        sh = out.at[idxk].get(mode='promise_in_bounds')
        out = out + jnp.where(iot >= k, sh, jnp.int32(0))
    return out


def _msgpass_call(src_t, dst_t, y, e_per_w, src_lim, dst_lim, v_rows):
    n_grp = e_per_w // 128
    stage = e_per_w + 256
    dump0 = e_per_w + 128
    ra = ACC_ROWS // NS

    @functools.partial(
        pl.kernel,
        out_type=jax.ShapeDtypeStruct((NC, N2, D), jnp.float32),
        mesh=_mesh(),
        scratch_types=[pltpu.VMEM((e_per_w,), jnp.int32),
                       pltpu.VMEM((e_per_w,), jnp.int32),
                       pltpu.VMEM((stage,), jnp.int32),
                       pltpu.VMEM((stage,), jnp.int32),
                       pltpu.VMEM((1, 128), jnp.int32),
                       pltpu.VMEM((128, D), jnp.float32),
                       pltpu.VMEM((ra, D), jnp.float32),
                       pltpu.VMEM_SHARED((ACC_ROWS, D), jnp.float32),
                       pltpu.VMEM_SHARED((NS * stage,), jnp.int32),
                       pltpu.VMEM_SHARED((NS * stage,), jnp.int32),
                       pltpu.SemaphoreType.DMA])
    def k(src_h, dst_h, y_h, accp_h,
          srcv, dstv, gsrc, gdst, offs_b, rows_v, stg, acc_s,
          gsrc_s, gdst_s, sem):
        c = lax.axis_index("c")
        s = lax.axis_index("s")
        w = s * NC + c
        iot = lax.iota(jnp.int32, 16)

        def zb(i, carry):
            r = i // (D // 16)
            col = (i % (D // 16)) * 16
            stg[r, pl.ds(col, 16)] = jnp.zeros((16,), jnp.float32)
            return carry
        lax.fori_loop(0, ra * D // 16, zb, 0)
        pltpu.sync_copy(stg, acc_s.at[pl.ds(s * ra, ra)])
        pltpu.sync_copy(src_h.at[w], srcv)
        pltpu.sync_copy(dst_h.at[w], dstv)
        plsc.subcore_barrier()

        def gbody(g, n):
            base = g * 128
            for kk in range(8):
                s16 = srcv[pl.ds(base + kk * 16, 16)]
                d16 = dstv[pl.ds(base + kk * 16, 16)]
                m = (s16 < src_lim) & (d16 < dst_lim)
                mi = jnp.where(m, jnp.int32(1), jnp.int32(0))
                cs = _psum16(mi)
                offs = jnp.where(m, (cs - mi) + n, dump0 + kk * 16 + iot)
                offs_b[0, pl.ds(kk * 16, 16)] = offs + s * stage
                n = n + cs[15]
            pltpu.sync_copy(srcv.at[pl.ds(base, 128)],
                            gsrc_s.at[offs_b.at[0]])
            pltpu.sync_copy(dstv.at[pl.ds(base, 128)],
                            gdst_s.at[offs_b.at[0]])
            return n
        n = lax.fori_loop(0, n_grp, gbody, jnp.int32(0))

        pltpu.sync_copy(gsrc_s.at[pl.ds(s * stage, stage)], gsrc)
        pltpu.sync_copy(gdst_s.at[pl.ds(s * stage, stage)], gdst)

        for kk in range(8):
            gsrc[pl.ds(n + kk * 16, 16)] = iot * (v_rows // 16) + kk * 7
            gdst[pl.ds(n + kk * 16, 16)] = jnp.full((16,), TRASH, jnp.int32) + s
        nch = (n + 127) >> 7

        def cbody(j, carry):
            pltpu.async_copy(
                y_h.at[gsrc.at[pl.ds(j * 128, 128)]], rows_v, sem).wait()
            for kk in range(8):
                didx = gdst[pl.ds(j * 128 + kk * 16, 16)]
                pltpu.sync_copy(rows_v.at[pl.ds(kk * 16, 16)],
                                acc_s.at[didx], add=True)
            return carry
        lax.fori_loop(0, nch, cbody, 0)
        plsc.subcore_barrier()
        ro = N2 // NS
        pltpu.sync_copy(acc_s.at[pl.ds(s * ro, ro)], stg.at[pl.ds(0, ro)])
        pltpu.sync_copy(stg.at[pl.ds(0, ro)], accp_h.at[c, pl.ds(s * ro, ro)])

    return k(src_t, dst_t, y)


def _tc_a(x8k, W1, deg1p):
    def body(x_ref, w_ref, d_ref, y_ref):
        dis = lax.rsqrt(d_ref[0, :] + d_ref[1, :] + 2.0)
        y_ref[...] = dis[:, None] * jnp.dot(
            x_ref[...], w_ref[...], preferred_element_type=jnp.float32)
    blk = 512
    return pl.pallas_call(
        body,
        grid=(N1 // blk,),
        in_specs=[pl.BlockSpec((blk, D), lambda i: (i, 0)),
                  pl.BlockSpec((D, D), lambda i: (0, 0)),
                  pl.BlockSpec((2, blk), lambda i: (0, i))],
        out_specs=pl.BlockSpec((blk, D), lambda i: (i, 0)),
        out_shape=jax.ShapeDtypeStruct((N1, D), jnp.float32))(x8k, W1, deg1p)


def _tc_b(a1p, y1k, d1k, b1, W2, d2p):
    def body(a_ref, y_ref, d1_ref, b_ref, w_ref, d2_ref, o_ref):
        dis1 = lax.rsqrt(d1_ref[0, :] + d1_ref[1, :] + 2.0)
        acc = a_ref[0] + a_ref[1] + 2.0 * y_ref[...]
        h = jnp.maximum(dis1[:, None] * acc + b_ref[...], 0.0)
        dis2 = lax.rsqrt(d2_ref[0, :] + d2_ref[1, :] + 1.0)
        o_ref[...] = dis2[:, None] * jnp.dot(
            h, w_ref[...], preferred_element_type=jnp.float32)
    return pl.pallas_call(
        body,
        out_shape=jax.ShapeDtypeStruct((N2, D), jnp.float32),
    )(a1p, y1k, d1k, b1, W2, d2p)


def _tc_c(a2p, y2, d2p, b2):
    def body(a_ref, y_ref, d_ref, b_ref, o_ref):
        dis2 = lax.rsqrt(d_ref[0, :] + d_ref[1, :] + 1.0)
        o_ref[...] = dis2[:, None] * (a_ref[0] + a_ref[1] + y_ref[...]) \
            + b_ref[...]
    return pl.pallas_call(
        body,
        out_shape=jax.ShapeDtypeStruct((N2, D), jnp.float32),
    )(a2p, y2, d2p, b2)


def kernel(x, edge_index1, edge_index2, W1, b1, W2, b2):
    ei1 = edge_index1.astype(jnp.int32)
    ei2 = edge_index2.astype(jnp.int32)
    src1, dst1 = ei1[0], ei1[1]
    src2, dst2 = ei2[0], ei2[1]
    x8k = x[:N1]

    ones_t = jnp.ones((E1 // NW // 128, 128), jnp.float32)

    deg1p, deg2p = _deg_call(
        dst1.reshape(NW, E1 // NW // 128, 128),
        dst2.reshape(NW, E2 // NW // 128, 128), ones_t)
    y1 = _tc_a(x8k, W1, deg1p)
    a1p = _msgpass_call(src1.reshape(NW, E1 // NW), dst1.reshape(NW, E1 // NW),
                        y1, E1 // NW, N1, N2, N1)
    y2 = _tc_b(a1p, y1[:N2], deg1p[:, :N2], b1.reshape(1, D), W2, deg2p)
    a2p = _msgpass_call(src2.reshape(NW, E2 // NW), dst2.reshape(NW, E2 // NW),
                        y2, E2 // NW, N2, N2, N2)
    return _tc_c(a2p, y2, deg2p, b2.reshape(1, D))
